# baseline (speedup 1.0000x reference)
import functools

import jax
import jax.numpy as jnp
from jax.experimental import pallas as pl
from jax.experimental.pallas import tpu as pltpu
from jax.experimental.pallas import tpu_sc as plsc

S = 2048
D = 768
H = 12
HD = 64
BS = 32
TK = 16
WS = 512
NBLK = S // BS
TQ = 256
NT = S // TQ
BPT = TQ // BS
SCALE = HD ** -0.5
KC = 1024
NKC = (BS * D) // KC
TA = 256
NTA = S // TA
WWIN = TA + WS


def _bf(x):
    return x.astype(jnp.bfloat16)


def _mm(a, b):
    return jax.lax.dot(_bf(a), _bf(b), preferred_element_type=jnp.float32)


def _mm_t(a, b):
    return jax.lax.dot_general(
        _bf(a), _bf(b), (((1,), (1,)), ((), ())),
        preferred_element_type=jnp.float32)


def _erf(z):
    a = jnp.abs(z)
    t = 1.0 / (1.0 + 0.3275911 * a)
    poly = t * (0.254829592 + t * (-0.284496736 + t * (1.421413741
               + t * (-1.453152027 + t * 1.061405429))))
    e = 1.0 - poly * jnp.exp(-a * a)
    return jnp.sign(z) * e


def _gelu(x):
    return x * 0.5 * (1.0 + _erf(x * (2.0 ** -0.5)))


def _proj_kernel(x_ref, wq_ref, bq_ref, wk_ref, bk_ref, wv_ref, bv_ref,
                 wg1_ref, bg1_ref, wg2_ref, bg2_ref,
                 q_ref, k_ref, v_ref, g_ref, idx_ref, kvt_ref,
                 qm_acc, kbm_acc):
    i = pl.program_id(0)
    x = x_ref[...]
    bsel = jnp.where(
        jax.lax.broadcasted_iota(jnp.int32, (BPT, TQ), 1) // BS
        == jax.lax.broadcasted_iota(jnp.int32, (BPT, TQ), 0),
        jnp.float32(1.0 / BS), 0.0)
    gsel = jnp.full((1, TQ), 1.0 / S, jnp.float32)
    q = _mm(x, wq_ref[...]) + bq_ref[...]
    k = _mm(x, wk_ref[...]) + bk_ref[...]
    v = _mm(x, wv_ref[...]) + bv_ref[...]
    qb, kb, vb = _bf(q), _bf(k), _bf(v)
    q_ref[...] = qb
    k_ref[...] = kb
    v_ref[...] = vb
    kvt_ref[...] = jnp.stack(
        [jnp.concatenate([kb[:, h * HD:(h + 1) * HD],
                          vb[:, h * HD:(h + 1) * HD]], axis=1)
         for h in range(H)], axis=0).astype(jnp.float32)
    g1 = _gelu(_mm(x, wg1_ref[...]) + bg1_ref[...])
    g_ref[...] = jax.nn.sigmoid(_mm(g1, wg2_ref[...]) + bg2_ref[...])
    qm = _mm(gsel, q)
    kbm = _mm(bsel, k)

    @pl.when(i == 0)
    def _():
        qm_acc[...] = jnp.zeros_like(qm_acc)

    qm_acc[...] += qm
    kbm_acc[pl.ds(i * BPT, BPT), :] = kbm

    @pl.when(i == NT - 1)
    def _():
        prod = kbm_acc[...] * qm_acc[...]
        hsel = (jax.lax.broadcasted_iota(jnp.int32, (D, 128), 0) // HD
                == jax.lax.broadcasted_iota(jnp.int32, (D, 128), 1))
        s = jax.lax.dot(prod, hsel.astype(jnp.float32),
                        precision=jax.lax.Precision.HIGHEST,
                        preferred_element_type=jnp.float32)
        sub = jax.lax.broadcasted_iota(jnp.int32, (NBLK, 128), 0)
        out = jnp.zeros((TK, 128), jnp.int32)
        row = jax.lax.broadcasted_iota(jnp.int32, (TK, 128), 0)
        for t in range(TK):
            m = jnp.max(s, axis=0, keepdims=True)
            eq = s >= m
            idx = jnp.min(jnp.where(eq, sub, NBLK), axis=0,
                          keepdims=True)
            s = jnp.where(sub == idx, -jnp.inf, s)
            out = jnp.where(row == t, idx, out)
        idx_ref[...] = out


def _compress_kernel(kr_ref, vr_ref, pos_ref, wc1_ref, bc1_ref,
                     wc2_ref, bc2_ref, kc_ref, vc_ref, acc):
    c = pl.program_id(0)
    pos = pos_ref[...]
    w1 = wc1_ref[...]
    hk = _mm(kr_ref[...].astype(jnp.float32) + pos, w1)
    hv = _mm(vr_ref[...].astype(jnp.float32) + pos, w1)

    @pl.when(c == 0)
    def _():
        acc[...] = jnp.zeros_like(acc)

    acc[pl.ds(0, NBLK), :] += hk
    acc[pl.ds(NBLK, NBLK), :] += hv

    @pl.when(c == NKC - 1)
    def _():
        h = _gelu(acc[...] + bc1_ref[...])
        out = _mm(h, wc2_ref[...]) + bc2_ref[...]
        kc_ref[...] = _bf(out[:NBLK])
        vc_ref[...] = _bf(out[NBLK:])


NSEL = H * TK * BS
GW = 128


def _sc_gather(x, rows):
    mesh = plsc.VectorSubcoreMesh(core_axis_name="c", subcore_axis_name="s")

    @functools.partial(
        pl.kernel,
        out_type=jax.ShapeDtypeStruct((NSEL, 2 * HD), jnp.float32),
        mesh=mesh)
    def gather_kernel(x_hbm, i_hbm, o_hbm):
        def body(i_vmem, o_vmem):
            pltpu.sync_copy(x_hbm.at[i_vmem.at[0]], o_vmem)

        pltpu.emit_pipeline(
            body,
            grid=(NSEL // GW,),
            in_specs=[pl.BlockSpec((1, GW), index_map=lambda i: (0, i))],
            out_specs=[pl.BlockSpec((GW, 2 * HD), index_map=lambda i: (i, 0))],
            core_axis_name="s",
            dimension_semantics=(pltpu.PARALLEL,),
        )(i_hbm, o_hbm)

    return gather_kernel(x, rows)


def _attn_kernel(q_ref, k_ref, v_ref, sel_ref, kc_ref, vc_ref, g_ref,
                 wo_ref, bo_ref, o_ref, selb):
    i = pl.program_id(0)
    t0 = i * TA
    start = pl.multiple_of(jnp.maximum(0, jnp.minimum(t0 - WS // 2, S - WWIN)),
                           TA)
    g = g_ref[...]
    g0 = g[:, 0:1]
    g1 = g[:, 1:2]
    g2 = g[:, 2:3]
    r = jax.lax.broadcasted_iota(jnp.int32, (TA, WWIN), 0)
    cidx = jax.lax.broadcasted_iota(jnp.int32, (TA, WWIN), 1)
    diff = (start + cidx) - (t0 + r)
    wmask = (diff >= -(WS // 2)) & (diff < WS // 2)

    @pl.when(i == 0)
    def _():
        selb[...] = _bf(sel_ref[...])

    SEL = TK * BS
    outs = []
    for h in range(H):
        hs = h * HD
        qh = q_ref[:, hs:hs + HD] * jnp.bfloat16(SCALE)
        ec = jnp.exp(_mm_t(qh, kc_ref[:, hs:hs + HD]))
        out_c = _mm(ec, vc_ref[:, hs:hs + HD])
        nc = g0 / jnp.sum(ec, axis=1, keepdims=True)
        es = jnp.exp(_mm_t(qh, selb[h * SEL:(h + 1) * SEL, 0:HD]))
        out_s = _mm(es, selb[h * SEL:(h + 1) * SEL, HD:2 * HD])
        ns = g1 / jnp.sum(es, axis=1, keepdims=True)
        kw = k_ref[pl.ds(start, WWIN), hs:hs + HD]
        vw = v_ref[pl.ds(start, WWIN), hs:hs + HD]
        ew = jnp.where(wmask, jnp.exp(_mm_t(qh, kw)), 0.0)
        out_w = _mm(ew, vw)
        nw = g2 / jnp.sum(ew, axis=1, keepdims=True)
        outs.append(nc * out_c + ns * out_s + nw * out_w)

    comb = jnp.concatenate(outs, axis=1)
    o_ref[...] = _mm(comb, wo_ref[...]) + bo_ref[...]


def kernel(hidden_states, Wq, bq, Wk, bk, Wv, bv, Wo, bo, pos_emb,
           Wc1, bc1, Wc2, bc2, Wg1, bg1, Wg2, bg2):
    B = hidden_states.shape[0]
    x = hidden_states.reshape(S, D)
    wg2p = jnp.zeros((D // 2, 128), jnp.float32).at[:, :3].set(Wg2)
    bg2p = jnp.zeros((1, 128), jnp.float32).at[:, :3].set(bg2)

    const = lambda bs: pl.BlockSpec(bs, lambda i: (0, 0))
    row = lambda bs: pl.BlockSpec(bs, lambda i: (i, 0))

    q, k, v, g, idx_out, kvt = pl.pallas_call(
        _proj_kernel,
        grid=(NT,),
        in_specs=[
            row((TQ, D)),
            const((D, D)), const((1, D)),
            const((D, D)), const((1, D)),
            const((D, D)), const((1, D)),
            const((D, D // 2)), const((1, D // 2)),
            const((D // 2, 128)), const((1, 128)),
        ],
        out_specs=[
            row((TQ, D)), row((TQ, D)), row((TQ, D)), row((TQ, 128)),
            const((TK, 128)),
            pl.BlockSpec((H, TQ, 2 * HD), lambda i: (0, i, 0)),
        ],
        out_shape=[
            jax.ShapeDtypeStruct((S, D), jnp.bfloat16),
            jax.ShapeDtypeStruct((S, D), jnp.bfloat16),
            jax.ShapeDtypeStruct((S, D), jnp.bfloat16),
            jax.ShapeDtypeStruct((S, 128), jnp.float32),
            jax.ShapeDtypeStruct((TK, 128), jnp.int32),
            jax.ShapeDtypeStruct((H, S, 2 * HD), jnp.float32),
        ],
        scratch_shapes=[
            pltpu.VMEM((1, D), jnp.float32),
            pltpu.VMEM((NBLK, D), jnp.float32),
        ],
    )(x, Wq, bq.reshape(1, D), Wk, bk.reshape(1, D), Wv, bv.reshape(1, D),
      Wg1, bg1.reshape(1, D // 2), wg2p, bg2p)

    top_idx = idx_out[:, :H].T.reshape(H * TK)

    hh = jnp.arange(H * TK, dtype=jnp.int32) // TK
    base = hh * S + top_idx * BS
    rows = (base[:, None] + jnp.arange(BS, dtype=jnp.int32)[None, :]
            ).reshape(1, NSEL)

    kv_sel = _sc_gather(kvt.reshape(H * S, 2 * HD), rows)

    k3 = k.reshape(NBLK, BS * D)
    v3 = v.reshape(NBLK, BS * D)
    pos3 = pos_emb.reshape(1, BS * D)

    kc, vc = pl.pallas_call(
        _compress_kernel,
        grid=(NKC,),
        in_specs=[
            pl.BlockSpec((NBLK, KC), lambda c: (0, c)),
            pl.BlockSpec((NBLK, KC), lambda c: (0, c)),
            pl.BlockSpec((1, KC), lambda c: (0, c)),
            pl.BlockSpec((KC, 4 * D), lambda c: (c, 0)),
            pl.BlockSpec((1, 4 * D), lambda c: (0, 0)),
            pl.BlockSpec((4 * D, D), lambda c: (0, 0)),
            pl.BlockSpec((1, D), lambda c: (0, 0)),
        ],
        out_specs=[
            pl.BlockSpec((NBLK, D), lambda c: (0, 0)),
            pl.BlockSpec((NBLK, D), lambda c: (0, 0)),
        ],
        out_shape=[
            jax.ShapeDtypeStruct((NBLK, D), jnp.bfloat16),
            jax.ShapeDtypeStruct((NBLK, D), jnp.bfloat16),
        ],
        scratch_shapes=[pltpu.VMEM((2 * NBLK, 4 * D), jnp.float32)],
    )(k3, v3, pos3, Wc1, bc1.reshape(1, 4 * D), Wc2, bc2.reshape(1, D))

    out = pl.pallas_call(
        _attn_kernel,
        grid=(NTA,),
        in_specs=[
            row((TA, D)),
            const((S, D)), const((S, D)),
            const((NSEL, 2 * HD)),
            const((NBLK, D)), const((NBLK, D)),
            row((TA, 128)),
            const((D, D)), const((1, D)),
        ],
        out_specs=row((TA, D)),
        out_shape=jax.ShapeDtypeStruct((S, D), jnp.float32),
        scratch_shapes=[pltpu.VMEM((NSEL, 2 * HD), jnp.bfloat16)],
    )(q, k, v, kv_sel, kc, vc, g, Wo, bo.reshape(1, D))

    return out.reshape(B, S, D)
